# baseline (speedup 1.0000x reference)
import jax
import jax.numpy as jnp
from jax.experimental import pallas as pl
from jax.experimental.pallas import tpu as pltpu

_SIGMA = 0.1
_NCORES = 2


def _round_up(v, m):
    return ((v + m - 1) // m) * m


def _weff_kernel(w_ref, nw_ref, b_ref, nb_ref, weff_ref, beff_ref):
    weff_ref[...] = (w_ref[...] + _SIGMA * nw_ref[...]).astype(jnp.bfloat16)
    beff_ref[...] = b_ref[...] + _SIGMA * nb_ref[...]


def _mm_kernel(x_ref, weff_ref, beff_ref, o_ref):
    o_ref[...] = (
        jnp.dot(x_ref[...].astype(jnp.bfloat16), weff_ref[...],
                preferred_element_type=jnp.float32)
        + beff_ref[...]
    )


def kernel(x, w_t, bias2d, noise_w_t, noise_b2d):
    B, K = x.shape
    Kw, N = w_t.shape
    assert K == Kw

    nh = N // _NCORES
    weff, beff = pl.pallas_call(
        _weff_kernel,
        grid=(_NCORES,),
        in_specs=[
            pl.BlockSpec((K, nh), lambda i: (0, i)),
            pl.BlockSpec((K, nh), lambda i: (0, i)),
            pl.BlockSpec((1, nh), lambda i: (0, i)),
            pl.BlockSpec((1, nh), lambda i: (0, i)),
        ],
        out_specs=[
            pl.BlockSpec((K, nh), lambda i: (0, i)),
            pl.BlockSpec((1, nh), lambda i: (0, i)),
        ],
        out_shape=[
            jax.ShapeDtypeStruct((K, N), jnp.bfloat16),
            jax.ShapeDtypeStruct((1, N), jnp.float32),
        ],
        compiler_params=pltpu.CompilerParams(
            dimension_semantics=("parallel",),
        ),
    )(w_t, noise_w_t, bias2d, noise_b2d)

    bt = _round_up(B, _NCORES) // _NCORES
    Bp = bt * _NCORES
    x_p = x if Bp == B else jnp.pad(x, ((0, Bp - B), (0, 0)))

    out = pl.pallas_call(
        _mm_kernel,
        grid=(_NCORES,),
        in_specs=[
            pl.BlockSpec((bt, K), lambda i: (i, 0)),
            pl.BlockSpec((K, N), lambda i: (0, 0)),
            pl.BlockSpec((1, N), lambda i: (0, 0)),
        ],
        out_specs=pl.BlockSpec((bt, N), lambda i: (i, 0)),
        out_shape=jax.ShapeDtypeStruct((Bp, N), jnp.float32),
        compiler_params=pltpu.CompilerParams(
            dimension_semantics=("parallel",),
            vmem_limit_bytes=48 << 20,
        ),
    )(x_p, weff, beff)

    return out if Bp == B else out[:B]


# speedup vs baseline: 1.1116x; 1.1116x over previous
import jax
import jax.numpy as jnp
from jax.experimental import pallas as pl
from jax.experimental.pallas import tpu as pltpu

_SIGMA = 0.1
_NCORES = 2


def _round_up(v, m):
    return ((v + m - 1) // m) * m


def _noise_linear_kernel(x_ref, w_ref, nw_ref, b_ref, nb_ref, o_ref):
    weff = (w_ref[...] + _SIGMA * nw_ref[...]).astype(jnp.bfloat16)
    beff = b_ref[...] + _SIGMA * nb_ref[...]
    o_ref[...] = (
        jnp.dot(x_ref[...].astype(jnp.bfloat16), weff,
                preferred_element_type=jnp.float32)
        + beff
    )


def kernel(x, w_t, bias2d, noise_w_t, noise_b2d):
    B, K = x.shape
    Kw, N = w_t.shape
    assert K == Kw

    bt = _round_up(B, 8 * _NCORES) // _NCORES
    Bp = bt * _NCORES
    x_p = x if Bp == B else jnp.pad(x, ((0, Bp - B), (0, 0)))

    out = pl.pallas_call(
        _noise_linear_kernel,
        grid=(_NCORES,),
        in_specs=[
            pl.BlockSpec((bt, K), lambda i: (i, 0)),
            pl.BlockSpec((K, N), lambda i: (0, 0)),
            pl.BlockSpec((K, N), lambda i: (0, 0)),
            pl.BlockSpec((1, N), lambda i: (0, 0)),
            pl.BlockSpec((1, N), lambda i: (0, 0)),
        ],
        out_specs=pl.BlockSpec((bt, N), lambda i: (i, 0)),
        out_shape=jax.ShapeDtypeStruct((Bp, N), jnp.float32),
        compiler_params=pltpu.CompilerParams(
            dimension_semantics=("parallel",),
            vmem_limit_bytes=48 << 20,
        ),
    )(x_p, w_t, noise_w_t, bias2d, noise_b2d)

    return out if Bp == B else out[:B]
